# Initial kernel scaffold; baseline (speedup 1.0000x reference)
import jax
import jax.numpy as jnp
from jax.experimental import pallas as pl


def kernel(x, edge_index, batch, gin_W, gin_b, proj_W1, proj_b1, proj_W2, proj_b2, pool_w):
    raise NotImplementedError("write your pallas kernel here")


# speedup vs baseline: 5.0617x; 5.0617x over previous
import functools

import jax
import jax.numpy as jnp
from jax import lax
from jax.experimental import pallas as pl
from jax.experimental.pallas import tpu as pltpu
from jax.experimental.pallas import tpu_sc as plsc

N = 10000
E = 320000
H = 128
NPAD = 10240
NC, NS = 2, 16
NW = NC * NS
CHUNK = 128
NCHUNK = 80
EPAD = NW * NCHUNK * CHUNK
ROWS_PER_TILE = NPAD // NS
OUT_CHUNK = 64

_f32 = jnp.float32


def _agg_body(x_hbm, src_hbm, dst_hbm, out_hbm,
              sidx_v, didx_v, rows_v, zero_v, copy_v, agg_sh, gsem):
    c = lax.axis_index("c")
    s = lax.axis_index("s")
    wid = s * NC + c

    for r in range(16):
        for j in range(H // 16):
            zero_v[r, pl.ds(j * 16, 16)] = jnp.zeros((16,), _f32)
    for i in range(ROWS_PER_TILE // 16):
        pltpu.sync_copy(zero_v, agg_sh.at[pl.ds(s * ROWS_PER_TILE + i * 16, 16)])

    pltpu.sync_copy(src_hbm.at[wid], sidx_v)
    pltpu.sync_copy(dst_hbm.at[wid], didx_v)
    plsc.subcore_barrier()

    def chunk_step(j, carry):
        pltpu.async_copy(x_hbm.at[sidx_v.at[j]], rows_v, gsem).wait()
        pltpu.sync_copy(rows_v, agg_sh.at[didx_v.at[j]], add=True)
        return carry

    lax.fori_loop(0, NCHUNK, chunk_step, 0, unroll=False)
    plsc.subcore_barrier()

    for i in range(ROWS_PER_TILE // OUT_CHUNK):
        base = s * ROWS_PER_TILE + i * OUT_CHUNK
        pltpu.sync_copy(agg_sh.at[pl.ds(base, OUT_CHUNK)], copy_v)
        pltpu.sync_copy(copy_v, out_hbm.at[c, pl.ds(base, OUT_CHUNK)])


@jax.jit
def _sc_agg(x_pad, src_r, dst_r):
    mesh = plsc.VectorSubcoreMesh(core_axis_name="c", subcore_axis_name="s")
    return pl.kernel(
        _agg_body,
        out_type=jax.ShapeDtypeStruct((NC, NPAD, H), _f32),
        mesh=mesh,
        scratch_types=[
            pltpu.VMEM((NCHUNK, CHUNK), jnp.int32),
            pltpu.VMEM((NCHUNK, CHUNK), jnp.int32),
            pltpu.VMEM((CHUNK, H), _f32),
            pltpu.VMEM((16, H), _f32),
            pltpu.VMEM((OUT_CHUNK, H), _f32),
            pltpu.VMEM_SHARED((NPAD, H), _f32),
            pltpu.SemaphoreType.DMA,
        ],
    )(x_pad, src_r, dst_r)


_BLK = 512


def _mlp_body(x_ref, a_ref, m_ref, w0_ref, b0_ref, w1_ref, b1_ref, o_ref):
    h = x_ref[...] + a_ref[0] + a_ref[1]
    h = jnp.maximum(
        jax.lax.dot_general(h, w0_ref[...], (((1,), (0,)), ((), ())),
                            precision=jax.lax.Precision.DEFAULT) + b0_ref[...], 0.0)
    h = jax.lax.dot_general(h, w1_ref[...], (((1,), (0,)), ((), ())),
                            precision=jax.lax.Precision.DEFAULT) + b1_ref[...]
    o_ref[...] = jnp.maximum(h, 0.0) * m_ref[...]


@jax.jit
def _tc_mlp(x_pad, agg, mask_col, W0, b0, W1, b1):
    grid = NPAD // _BLK
    return pl.pallas_call(
        _mlp_body,
        grid=(grid,),
        in_specs=[
            pl.BlockSpec((_BLK, H), lambda i: (i, 0)),
            pl.BlockSpec((NC, _BLK, H), lambda i: (0, i, 0)),
            pl.BlockSpec((_BLK, 1), lambda i: (i, 0)),
            pl.BlockSpec((H, H), lambda i: (0, 0)),
            pl.BlockSpec((1, H), lambda i: (0, 0)),
            pl.BlockSpec((H, H), lambda i: (0, 0)),
            pl.BlockSpec((1, H), lambda i: (0, 0)),
        ],
        out_specs=pl.BlockSpec((_BLK, H), lambda i: (i, 0)),
        out_shape=jax.ShapeDtypeStruct((NPAD, H), _f32),
    )(x_pad, agg, mask_col, W0, b0[None, :], W1, b1[None, :])


def _epilogue_body(k, x_ref, m_ref, w_ref, p1_ref, c1_ref, p2_ref, c2_ref,
                   g0_ref, g1_ref,
                   xo_ref, mo_ref, gs_ref, pj_ref, out_ref):
    x = x_ref[...]
    alive = m_ref[...] > 0.5
    kf = jnp.float32(k)

    p = jnp.sum(x, axis=0, keepdims=True)
    ph = jnp.maximum(
        jax.lax.dot_general(p, p1_ref[...], (((1,), (0,)), ((), ())),
                            precision=jax.lax.Precision.DEFAULT) + c1_ref[...], 0.0)
    pj_ref[...] = jax.lax.dot_general(ph, p2_ref[...], (((1,), (0,)), ((), ())),
                                      precision=jax.lax.Precision.DEFAULT) + c2_ref[...]

    w = w_ref[...]
    inv_norm = jax.lax.rsqrt(jnp.sum(w * w))
    score = jax.lax.dot_general(x, w, (((1,), (0,)), ((), ())),
                                precision=jax.lax.Precision.DEFAULT) * inv_norm

    s = jax.lax.bitcast_convert_type(score, jnp.int32)
    key = s ^ jnp.where(s < 0, jnp.int32(0x7FFFFFFF), jnp.int32(0))

    def cnt_ge(t):
        return jnp.sum(jnp.where(alive & (key >= t), 1.0, 0.0))

    T0 = jnp.where(cnt_ge(jnp.int32(0)) >= kf, jnp.int32(0), jnp.int32(-(2**31)))

    def t_step(i, t):
        cand = t | (jnp.int32(1) << (30 - i))
        return jnp.where(cnt_ge(cand) >= kf, cand, t)

    T = lax.fori_loop(0, 31, t_step, T0)

    gt = alive & (key > T)
    eq = alive & (key == T)
    n_gt = jnp.sum(jnp.where(gt, 1.0, 0.0))
    need_eq = kf - n_gt

    idx = jax.lax.broadcasted_iota(jnp.int32, (NPAD, 1), 0)

    def i_step(i, t):
        cand = t | (jnp.int32(1) << (13 - i))
        cnt = jnp.sum(jnp.where(eq & (idx < cand), 1.0, 0.0))
        return jnp.where(cnt < need_eq, cand, t)

    tmax = lax.fori_loop(0, 14, i_step, jnp.int32(0))
    keep = gt | (eq & (idx <= tmax) & (need_eq > 0.0))

    gate = jnp.tanh(score)
    xg = jnp.where(keep, x * gate, 0.0)
    gmax = jnp.max(jnp.where(keep, xg, -jnp.inf), axis=0, keepdims=True)
    gmean = jnp.sum(xg, axis=0, keepdims=True) / kf
    gs = jnp.concatenate([gmax, gmean], axis=1)

    xo_ref[...] = xg
    mo_ref[...] = jnp.where(keep, 1.0, 0.0)
    gs_ref[...] = gs
    out_ref[...] = (jnp.maximum(g0_ref[...], 0.0) + jnp.maximum(g1_ref[...], 0.0)
                    + jnp.maximum(gs, 0.0))


@functools.partial(jax.jit, static_argnums=(0,))
def _tc_epilogue(k, x_pad, mask_col, pool_w, P1, c1, P2, c2, gs0, gs1):
    return pl.pallas_call(
        functools.partial(_epilogue_body, k),
        out_shape=(
            jax.ShapeDtypeStruct((NPAD, H), _f32),
            jax.ShapeDtypeStruct((NPAD, 1), _f32),
            jax.ShapeDtypeStruct((1, 2 * H), _f32),
            jax.ShapeDtypeStruct((1, H), _f32),
            jax.ShapeDtypeStruct((1, 2 * H), _f32),
        ),
    )(x_pad, mask_col, pool_w[:, None], P1, c1[None, :], P2, c2[None, :], gs0, gs1)


def kernel(x, edge_index, batch, gin_W, gin_b, proj_W1, proj_b1, proj_W2,
           proj_b2, pool_w):
    src = edge_index[0]
    dst = edge_index[1]
    pad_id = jnp.int32(NPAD - 1)
    src_r = jnp.concatenate(
        [src, jnp.full((EPAD - E,), pad_id, jnp.int32)]).reshape(NW, NCHUNK, CHUNK)
    dst_r = jnp.concatenate(
        [dst, jnp.full((EPAD - E,), pad_id, jnp.int32)]).reshape(NW, NCHUNK, CHUNK)

    h = jnp.zeros((NPAD, H), _f32).at[:N].set(x)
    mask = (jnp.arange(NPAD, dtype=jnp.int32) < N).astype(_f32)[:, None]

    ks = [5000, 2500, 1250]
    gs_list = []
    proj_list = []
    zero_gs = jnp.zeros((1, 2 * H), _f32)
    out = None
    for i in range(3):
        for l in range(2):
            agg = _sc_agg(h, src_r, dst_r)
            h = _tc_mlp(h, agg, mask, gin_W[i, l, 0], gin_b[i, l, 0],
                        gin_W[i, l, 1], gin_b[i, l, 1])
        g0 = gs_list[0] if i == 2 else zero_gs
        g1 = gs_list[1] if i == 2 else zero_gs
        h, mask, gs, pj, out = _tc_epilogue(
            ks[i], h, mask, pool_w[i], proj_W1[i], proj_b1[i], proj_W2[i],
            proj_b2[i], g0, g1)
        gs_list.append(gs)
        proj_list.append(pj)

    return (out, gs_list[0], gs_list[1], gs_list[2],
            proj_list[0], proj_list[1], proj_list[2])


# speedup vs baseline: 5.0687x; 1.0014x over previous
import functools

import jax
import jax.numpy as jnp
from jax import lax
from jax.experimental import pallas as pl
from jax.experimental.pallas import tpu as pltpu
from jax.experimental.pallas import tpu_sc as plsc

N = 10000
E = 320000
H = 128
NPAD = 10240
NC, NS = 2, 16
NW = NC * NS
CHUNK = 128
NCHUNK = 80
EPAD = NW * NCHUNK * CHUNK
ROWS_PER_TILE = NPAD // NS
OUT_CHUNK = 64
NBUF = 2

_f32 = jnp.float32


def _agg_body(x_hbm, src_hbm, dst_hbm, out_hbm,
              sidx_v, didx_v, rows_v, zero_v, copy_v, agg_sh, isem, gsem):
    c = lax.axis_index("c")
    s = lax.axis_index("s")
    wid = s * NC + c

    for r in range(16):
        for j in range(H // 16):
            zero_v[r, pl.ds(j * 16, 16)] = jnp.zeros((16,), _f32)
    for i in range(ROWS_PER_TILE // 16):
        pltpu.sync_copy(zero_v, agg_sh.at[pl.ds(s * ROWS_PER_TILE + i * 16, 16)])

    def idx_start(j, b):
        pltpu.async_copy(src_hbm.at[wid, j], sidx_v.at[b], isem.at[b])
        pltpu.async_copy(dst_hbm.at[wid, j], didx_v.at[b], isem.at[b])

    def idx_wait(j, b):
        pltpu.make_async_copy(src_hbm.at[wid, j], sidx_v.at[b], isem.at[b]).wait()
        pltpu.make_async_copy(dst_hbm.at[wid, j], didx_v.at[b], isem.at[b]).wait()

    def gather_start(b):
        pltpu.async_copy(x_hbm.at[sidx_v.at[b]], rows_v.at[b], gsem.at[b])

    def gather_wait(b):
        pltpu.make_async_copy(x_hbm.at[sidx_v.at[b]], rows_v.at[b],
                              gsem.at[b]).wait()

    plsc.subcore_barrier()

    idx_start(0, 0)
    idx_start(1, 1)
    idx_wait(0, 0)
    gather_start(0)

    def pair_step(j0, carry):
        j = j0 * 2
        gather_wait(0)
        pltpu.sync_copy(rows_v.at[0], agg_sh.at[didx_v.at[0]], add=True)

        @pl.when(j + 2 < NCHUNK)
        def _():
            idx_start(j + 2, 0)

        idx_wait(j + 1, 1)
        gather_start(1)

        gather_wait(1)
        pltpu.sync_copy(rows_v.at[1], agg_sh.at[didx_v.at[1]], add=True)

        @pl.when(j + 3 < NCHUNK)
        def _():
            idx_start(j + 3, 1)

        @pl.when(j + 2 < NCHUNK)
        def _():
            idx_wait(j + 2, 0)
            gather_start(0)

        return carry

    lax.fori_loop(0, NCHUNK // 2, pair_step, 0, unroll=False)
    plsc.subcore_barrier()

    for i in range(ROWS_PER_TILE // OUT_CHUNK):
        base = s * ROWS_PER_TILE + i * OUT_CHUNK
        pltpu.sync_copy(agg_sh.at[pl.ds(base, OUT_CHUNK)], copy_v)
        pltpu.sync_copy(copy_v, out_hbm.at[c, pl.ds(base, OUT_CHUNK)])


@jax.jit
def _sc_agg(x_pad, src_r, dst_r):
    mesh = plsc.VectorSubcoreMesh(core_axis_name="c", subcore_axis_name="s")
    return pl.kernel(
        _agg_body,
        out_type=jax.ShapeDtypeStruct((NC, NPAD, H), _f32),
        mesh=mesh,
        scratch_types=[
            pltpu.VMEM((NBUF, CHUNK), jnp.int32),
            pltpu.VMEM((NBUF, CHUNK), jnp.int32),
            pltpu.VMEM((NBUF, CHUNK, H), _f32),
            pltpu.VMEM((16, H), _f32),
            pltpu.VMEM((OUT_CHUNK, H), _f32),
            pltpu.VMEM_SHARED((NPAD, H), _f32),
            pltpu.SemaphoreType.DMA((NBUF,)),
            pltpu.SemaphoreType.DMA((NBUF,)),
        ],
    )(x_pad, src_r, dst_r)


_BLK = 512


def _mlp_body(x_ref, a_ref, m_ref, w0_ref, b0_ref, w1_ref, b1_ref, o_ref):
    h = x_ref[...] + a_ref[0] + a_ref[1]
    h = jnp.maximum(
        jax.lax.dot_general(h, w0_ref[...], (((1,), (0,)), ((), ())),
                            precision=jax.lax.Precision.DEFAULT) + b0_ref[...], 0.0)
    h = jax.lax.dot_general(h, w1_ref[...], (((1,), (0,)), ((), ())),
                            precision=jax.lax.Precision.DEFAULT) + b1_ref[...]
    o_ref[...] = jnp.maximum(h, 0.0) * m_ref[...]


@jax.jit
def _tc_mlp(x_pad, agg, mask_col, W0, b0, W1, b1):
    grid = NPAD // _BLK
    return pl.pallas_call(
        _mlp_body,
        grid=(grid,),
        in_specs=[
            pl.BlockSpec((_BLK, H), lambda i: (i, 0)),
            pl.BlockSpec((NC, _BLK, H), lambda i: (0, i, 0)),
            pl.BlockSpec((_BLK, 1), lambda i: (i, 0)),
            pl.BlockSpec((H, H), lambda i: (0, 0)),
            pl.BlockSpec((1, H), lambda i: (0, 0)),
            pl.BlockSpec((H, H), lambda i: (0, 0)),
            pl.BlockSpec((1, H), lambda i: (0, 0)),
        ],
        out_specs=pl.BlockSpec((_BLK, H), lambda i: (i, 0)),
        out_shape=jax.ShapeDtypeStruct((NPAD, H), _f32),
    )(x_pad, agg, mask_col, W0, b0[None, :], W1, b1[None, :])


def _epilogue_body(k, x_ref, m_ref, w_ref, p1_ref, c1_ref, p2_ref, c2_ref,
                   g0_ref, g1_ref,
                   xo_ref, mo_ref, gs_ref, pj_ref, out_ref):
    x = x_ref[...]
    alive = m_ref[...] > 0.5
    kf = jnp.float32(k)

    p = jnp.sum(x, axis=0, keepdims=True)
    ph = jnp.maximum(
        jax.lax.dot_general(p, p1_ref[...], (((1,), (0,)), ((), ())),
                            precision=jax.lax.Precision.DEFAULT) + c1_ref[...], 0.0)
    pj_ref[...] = jax.lax.dot_general(ph, p2_ref[...], (((1,), (0,)), ((), ())),
                                      precision=jax.lax.Precision.DEFAULT) + c2_ref[...]

    w = w_ref[...]
    inv_norm = jax.lax.rsqrt(jnp.sum(w * w))
    score = jax.lax.dot_general(x, w, (((1,), (0,)), ((), ())),
                                precision=jax.lax.Precision.DEFAULT) * inv_norm

    s = jax.lax.bitcast_convert_type(score, jnp.int32)
    key = s ^ jnp.where(s < 0, jnp.int32(0x7FFFFFFF), jnp.int32(0))

    def cnt_ge(t):
        return jnp.sum(jnp.where(alive & (key >= t), 1.0, 0.0))

    T0 = jnp.where(cnt_ge(jnp.int32(0)) >= kf, jnp.int32(0), jnp.int32(-(2**31)))

    def t_step(i, t):
        cand = t | (jnp.int32(1) << (30 - i))
        return jnp.where(cnt_ge(cand) >= kf, cand, t)

    T = lax.fori_loop(0, 31, t_step, T0)

    gt = alive & (key > T)
    eq = alive & (key == T)
    n_gt = jnp.sum(jnp.where(gt, 1.0, 0.0))
    need_eq = kf - n_gt

    idx = jax.lax.broadcasted_iota(jnp.int32, (NPAD, 1), 0)

    def i_step(i, t):
        cand = t | (jnp.int32(1) << (13 - i))
        cnt = jnp.sum(jnp.where(eq & (idx < cand), 1.0, 0.0))
        return jnp.where(cnt < need_eq, cand, t)

    tmax = lax.fori_loop(0, 14, i_step, jnp.int32(0))
    keep = gt | (eq & (idx <= tmax) & (need_eq > 0.0))

    gate = jnp.tanh(score)
    xg = jnp.where(keep, x * gate, 0.0)
    gmax = jnp.max(jnp.where(keep, xg, -jnp.inf), axis=0, keepdims=True)
    gmean = jnp.sum(xg, axis=0, keepdims=True) / kf
    gs = jnp.concatenate([gmax, gmean], axis=1)

    xo_ref[...] = xg
    mo_ref[...] = jnp.where(keep, 1.0, 0.0)
    gs_ref[...] = gs
    out_ref[...] = (jnp.maximum(g0_ref[...], 0.0) + jnp.maximum(g1_ref[...], 0.0)
                    + jnp.maximum(gs, 0.0))


@functools.partial(jax.jit, static_argnums=(0,))
def _tc_epilogue(k, x_pad, mask_col, pool_w, P1, c1, P2, c2, gs0, gs1):
    return pl.pallas_call(
        functools.partial(_epilogue_body, k),
        out_shape=(
            jax.ShapeDtypeStruct((NPAD, H), _f32),
            jax.ShapeDtypeStruct((NPAD, 1), _f32),
            jax.ShapeDtypeStruct((1, 2 * H), _f32),
            jax.ShapeDtypeStruct((1, H), _f32),
            jax.ShapeDtypeStruct((1, 2 * H), _f32),
        ),
    )(x_pad, mask_col, pool_w[:, None], P1, c1[None, :], P2, c2[None, :], gs0, gs1)


def kernel(x, edge_index, batch, gin_W, gin_b, proj_W1, proj_b1, proj_W2,
           proj_b2, pool_w):
    src = edge_index[0]
    dst = edge_index[1]
    pad_id = jnp.int32(NPAD - 1)
    src_r = jnp.concatenate(
        [src, jnp.full((EPAD - E,), pad_id, jnp.int32)]).reshape(NW, NCHUNK, CHUNK)
    dst_r = jnp.concatenate(
        [dst, jnp.full((EPAD - E,), pad_id, jnp.int32)]).reshape(NW, NCHUNK, CHUNK)

    h = jnp.zeros((NPAD, H), _f32).at[:N].set(x)
    mask = (jnp.arange(NPAD, dtype=jnp.int32) < N).astype(_f32)[:, None]

    ks = [5000, 2500, 1250]
    gs_list = []
    proj_list = []
    zero_gs = jnp.zeros((1, 2 * H), _f32)
    out = None
    for i in range(3):
        for l in range(2):
            agg = _sc_agg(h, src_r, dst_r)
            h = _tc_mlp(h, agg, mask, gin_W[i, l, 0], gin_b[i, l, 0],
                        gin_W[i, l, 1], gin_b[i, l, 1])
        g0 = gs_list[0] if i == 2 else zero_gs
        g1 = gs_list[1] if i == 2 else zero_gs
        h, mask, gs, pj, out = _tc_epilogue(
            ks[i], h, mask, pool_w[i], proj_W1[i], proj_b1[i], proj_W2[i],
            proj_b2[i], g0, g1)
        gs_list.append(gs)
        proj_list.append(pj)

    return (out, gs_list[0], gs_list[1], gs_list[2],
            proj_list[0], proj_list[1], proj_list[2])


# speedup vs baseline: 6.9516x; 1.3715x over previous
import functools

import jax
import jax.numpy as jnp
from jax import lax
from jax.experimental import pallas as pl
from jax.experimental.pallas import tpu as pltpu
from jax.experimental.pallas import tpu_sc as plsc

N = 10000
E = 320000
H = 128
NPAD = 10240
NC, NS = 2, 16
NW = NC * NS
CHUNK = 128
NCHUNK = 80
EPAD = NW * NCHUNK * CHUNK
ROWS_PER_TILE = NPAD // NS
OUT_CHUNK = 64
NBUF = 2

_f32 = jnp.float32


def _agg_body(x_hbm, src_hbm, dst_hbm, cnt_hbm, out_hbm,
              sidx_v, didx_v, rows_v, zero_v, copy_v, cnt_v, agg_sh,
              isem, gsem):
    c = lax.axis_index("c")
    s = lax.axis_index("s")
    wid = s * NC + c

    pltpu.sync_copy(cnt_hbm.at[wid], cnt_v)
    nch = lax.reduce_max(cnt_v[...], axes=(0,))

    for r in range(16):
        for j in range(H // 16):
            zero_v[r, pl.ds(j * 16, 16)] = jnp.zeros((16,), _f32)
    for i in range(ROWS_PER_TILE // 16):
        pltpu.sync_copy(zero_v, agg_sh.at[pl.ds(s * ROWS_PER_TILE + i * 16, 16)])

    def idx_start(j, b):
        pltpu.async_copy(src_hbm.at[wid, j], sidx_v.at[b], isem.at[b])
        pltpu.async_copy(dst_hbm.at[wid, j], didx_v.at[b], isem.at[b])

    def idx_wait(j, b):
        pltpu.make_async_copy(src_hbm.at[wid, j], sidx_v.at[b], isem.at[b]).wait()
        pltpu.make_async_copy(dst_hbm.at[wid, j], didx_v.at[b], isem.at[b]).wait()

    def gather_start(b):
        pltpu.async_copy(x_hbm.at[sidx_v.at[b]], rows_v.at[b], gsem.at[b])

    def gather_wait(b):
        pltpu.make_async_copy(x_hbm.at[sidx_v.at[b]], rows_v.at[b],
                              gsem.at[b]).wait()

    plsc.subcore_barrier()

    idx_start(0, 0)
    idx_start(1, 1)
    idx_wait(0, 0)
    gather_start(0)

    def pair_step(j0, carry):
        j = j0 * 2
        gather_wait(0)
        pltpu.sync_copy(rows_v.at[0], agg_sh.at[didx_v.at[0]], add=True)

        @pl.when(j + 2 < nch)
        def _():
            idx_start(j + 2, 0)

        idx_wait(j + 1, 1)
        gather_start(1)

        gather_wait(1)
        pltpu.sync_copy(rows_v.at[1], agg_sh.at[didx_v.at[1]], add=True)

        @pl.when(j + 3 < nch)
        def _():
            idx_start(j + 3, 1)

        @pl.when(j + 2 < nch)
        def _():
            idx_wait(j + 2, 0)
            gather_start(0)

        return carry

    lax.fori_loop(0, nch // 2, pair_step, 0, unroll=False)
    plsc.subcore_barrier()

    for i in range(ROWS_PER_TILE // OUT_CHUNK):
        base = s * ROWS_PER_TILE + i * OUT_CHUNK
        pltpu.sync_copy(agg_sh.at[pl.ds(base, OUT_CHUNK)], copy_v)
        pltpu.sync_copy(copy_v, out_hbm.at[c, pl.ds(base, OUT_CHUNK)])


@jax.jit
def _sc_agg(x_pad, src_r, dst_r, counts):
    mesh = plsc.VectorSubcoreMesh(core_axis_name="c", subcore_axis_name="s")
    return pl.kernel(
        _agg_body,
        compiler_params=pltpu.CompilerParams(needs_layout_passes=False),
        out_type=jax.ShapeDtypeStruct((NC, NPAD, H), _f32),
        mesh=mesh,
        scratch_types=[
            pltpu.VMEM((NBUF, CHUNK), jnp.int32),
            pltpu.VMEM((NBUF, CHUNK), jnp.int32),
            pltpu.VMEM((NBUF, CHUNK, H), _f32),
            pltpu.VMEM((16, H), _f32),
            pltpu.VMEM((OUT_CHUNK, H), _f32),
            pltpu.VMEM((16,), jnp.int32),
            pltpu.VMEM_SHARED((NPAD, H), _f32),
            pltpu.SemaphoreType.DMA((NBUF,)),
            pltpu.SemaphoreType.DMA((NBUF,)),
        ],
    )(x_pad, src_r, dst_r, counts)


_PADV = NPAD - 1
_VPC = CHUNK // 16


def _filter_body(src_hbm, dst_hbm, keep_hbm, cin_hbm, osrc_hbm, odst_hbm,
                 cnt_hbm, keep_v, sidx_v, didx_v, osrc_v, odst_v, cnt_v,
                 cin_v, pad_v):
    c = lax.axis_index("c")
    s = lax.axis_index("s")
    wid = s * NC + c

    pltpu.sync_copy(cin_hbm.at[wid], cin_v)
    nch_in = lax.reduce_max(cin_v[...], axes=(0,))
    pltpu.sync_copy(keep_hbm, keep_v)
    pltpu.sync_copy(src_hbm.at[wid], sidx_v)
    pltpu.sync_copy(dst_hbm.at[wid], didx_v)

    for i in range(16):
        pad_v[pl.ds(i * 16, 16)] = jnp.full((16,), _PADV, jnp.int32)

    def step(i, carry):
        cnt, cnt_vec = carry
        sv = sidx_v[pl.ds(i * 16, 16)]
        dv = didx_v[pl.ds(i * 16, 16)]
        ks = plsc.load_gather(keep_v, [sv >> 7, sv & 127])
        kd = plsc.load_gather(keep_v, [dv >> 7, dv & 127])
        m = (ks > 0.5) & (kd > 0.5)
        plsc.store_compressed(osrc_v.at[pl.ds(cnt, 16)], sv, mask=m)
        plsc.store_compressed(odst_v.at[pl.ds(cnt, 16)], dv, mask=m)
        npop = plsc.all_reduce_population_count(m)
        return cnt + lax.reduce_max(npop, axes=(0,)), cnt_vec + npop

    cnt, cnt_vec = lax.fori_loop(
        0, nch_in * _VPC, step,
        (jnp.int32(0), jnp.zeros((16,), jnp.int32)), unroll=False)

    def pad_step(i, carry):
        base = cnt + i * 16
        osrc_v[pl.ds(base, 16)] = pad_v[pl.ds(0, 16)]
        odst_v[pl.ds(base, 16)] = pad_v[pl.ds(0, 16)]
        return carry

    lax.fori_loop(0, 17, pad_step, 0, unroll=False)

    cnt_v[...] = jnp.maximum((cnt_vec + 255) // 256 * 2, 2)
    pltpu.sync_copy(cnt_v, cnt_hbm.at[wid])
    pltpu.sync_copy(osrc_v.at[pl.ds(0, NCHUNK * CHUNK)], osrc_hbm.at[wid])
    pltpu.sync_copy(odst_v.at[pl.ds(0, NCHUNK * CHUNK)], odst_hbm.at[wid])


@jax.jit
def _sc_filter(src_r, dst_r, keep, counts_in):
    mesh = plsc.VectorSubcoreMesh(core_axis_name="c", subcore_axis_name="s")
    src_f = src_r.reshape(NW, NCHUNK * CHUNK)
    dst_f = dst_r.reshape(NW, NCHUNK * CHUNK)
    osrc, odst, counts = pl.kernel(
        _filter_body,
        compiler_params=pltpu.CompilerParams(use_tc_tiling_on_sc=False,
                                             needs_layout_passes=False),
        out_type=(
            jax.ShapeDtypeStruct((NW, NCHUNK * CHUNK), jnp.int32),
            jax.ShapeDtypeStruct((NW, NCHUNK * CHUNK), jnp.int32),
            jax.ShapeDtypeStruct((NW, 16), jnp.int32),
        ),
        mesh=mesh,
        scratch_types=[
            pltpu.VMEM((NPAD // 128, 128), _f32),
            pltpu.VMEM((NCHUNK * CHUNK,), jnp.int32),
            pltpu.VMEM((NCHUNK * CHUNK,), jnp.int32),
            pltpu.VMEM((NCHUNK * CHUNK + 512,), jnp.int32),
            pltpu.VMEM((NCHUNK * CHUNK + 512,), jnp.int32),
            pltpu.VMEM((16,), jnp.int32),
            pltpu.VMEM((16,), jnp.int32),
            pltpu.VMEM((256,), jnp.int32),
        ],
    )(src_f, dst_f, keep.reshape(NPAD // 128, 128), counts_in)
    return (osrc.reshape(NW, NCHUNK, CHUNK), odst.reshape(NW, NCHUNK, CHUNK),
            counts)


_BLK = 512


def _mlp_body(x_ref, a_ref, m_ref, w0_ref, b0_ref, w1_ref, b1_ref, o_ref):
    h = x_ref[...] + a_ref[0] + a_ref[1]
    h = jnp.maximum(
        jax.lax.dot_general(h, w0_ref[...], (((1,), (0,)), ((), ())),
                            precision=jax.lax.Precision.DEFAULT) + b0_ref[...], 0.0)
    h = jax.lax.dot_general(h, w1_ref[...], (((1,), (0,)), ((), ())),
                            precision=jax.lax.Precision.DEFAULT) + b1_ref[...]
    o_ref[...] = jnp.maximum(h, 0.0) * m_ref[...]


@jax.jit
def _tc_mlp(x_pad, agg, mask_col, W0, b0, W1, b1):
    grid = NPAD // _BLK
    return pl.pallas_call(
        _mlp_body,
        grid=(grid,),
        in_specs=[
            pl.BlockSpec((_BLK, H), lambda i: (i, 0)),
            pl.BlockSpec((NC, _BLK, H), lambda i: (0, i, 0)),
            pl.BlockSpec((_BLK, 1), lambda i: (i, 0)),
            pl.BlockSpec((H, H), lambda i: (0, 0)),
            pl.BlockSpec((1, H), lambda i: (0, 0)),
            pl.BlockSpec((H, H), lambda i: (0, 0)),
            pl.BlockSpec((1, H), lambda i: (0, 0)),
        ],
        out_specs=pl.BlockSpec((_BLK, H), lambda i: (i, 0)),
        out_shape=jax.ShapeDtypeStruct((NPAD, H), _f32),
    )(x_pad, agg, mask_col, W0, b0[None, :], W1, b1[None, :])


def _epilogue_body(k, x_ref, m_ref, w_ref, p1_ref, c1_ref, p2_ref, c2_ref,
                   g0_ref, g1_ref,
                   xo_ref, mo_ref, gs_ref, pj_ref, out_ref):
    x = x_ref[...]
    alive = m_ref[...] > 0.5
    kf = jnp.float32(k)

    p = jnp.sum(x, axis=0, keepdims=True)
    ph = jnp.maximum(
        jax.lax.dot_general(p, p1_ref[...], (((1,), (0,)), ((), ())),
                            precision=jax.lax.Precision.DEFAULT) + c1_ref[...], 0.0)
    pj_ref[...] = jax.lax.dot_general(ph, p2_ref[...], (((1,), (0,)), ((), ())),
                                      precision=jax.lax.Precision.DEFAULT) + c2_ref[...]

    w = w_ref[...]
    inv_norm = jax.lax.rsqrt(jnp.sum(w * w))
    score = jax.lax.dot_general(x, w, (((1,), (0,)), ((), ())),
                                precision=jax.lax.Precision.DEFAULT) * inv_norm

    s = jax.lax.bitcast_convert_type(score, jnp.int32)
    key = s ^ jnp.where(s < 0, jnp.int32(0x7FFFFFFF), jnp.int32(0))

    def cnt_ge(t):
        return jnp.sum(jnp.where(alive & (key >= t), 1.0, 0.0))

    T0 = jnp.where(cnt_ge(jnp.int32(0)) >= kf, jnp.int32(0), jnp.int32(-(2**31)))

    def t_step(i, t):
        cand = t | (jnp.int32(1) << (30 - i))
        return jnp.where(cnt_ge(cand) >= kf, cand, t)

    T = lax.fori_loop(0, 31, t_step, T0)

    gt = alive & (key > T)
    eq = alive & (key == T)
    n_gt = jnp.sum(jnp.where(gt, 1.0, 0.0))
    need_eq = kf - n_gt

    idx = jax.lax.broadcasted_iota(jnp.int32, (NPAD, 1), 0)

    def i_step(i, t):
        cand = t | (jnp.int32(1) << (13 - i))
        cnt = jnp.sum(jnp.where(eq & (idx < cand), 1.0, 0.0))
        return jnp.where(cnt < need_eq, cand, t)

    tmax = lax.fori_loop(0, 14, i_step, jnp.int32(0))
    keep = gt | (eq & (idx <= tmax) & (need_eq > 0.0))

    gate = jnp.tanh(score)
    xg = jnp.where(keep, x * gate, 0.0)
    gmax = jnp.max(jnp.where(keep, xg, -jnp.inf), axis=0, keepdims=True)
    gmean = jnp.sum(xg, axis=0, keepdims=True) / kf
    gs = jnp.concatenate([gmax, gmean], axis=1)

    xo_ref[...] = xg
    mo_ref[...] = jnp.where(keep, 1.0, 0.0)
    gs_ref[...] = gs
    out_ref[...] = (jnp.maximum(g0_ref[...], 0.0) + jnp.maximum(g1_ref[...], 0.0)
                    + jnp.maximum(gs, 0.0))


@functools.partial(jax.jit, static_argnums=(0,))
def _tc_epilogue(k, x_pad, mask_col, pool_w, P1, c1, P2, c2, gs0, gs1):
    return pl.pallas_call(
        functools.partial(_epilogue_body, k),
        out_shape=(
            jax.ShapeDtypeStruct((NPAD, H), _f32),
            jax.ShapeDtypeStruct((NPAD, 1), _f32),
            jax.ShapeDtypeStruct((1, 2 * H), _f32),
            jax.ShapeDtypeStruct((1, H), _f32),
            jax.ShapeDtypeStruct((1, 2 * H), _f32),
        ),
    )(x_pad, mask_col, pool_w[:, None], P1, c1[None, :], P2, c2[None, :], gs0, gs1)


def kernel(x, edge_index, batch, gin_W, gin_b, proj_W1, proj_b1, proj_W2,
           proj_b2, pool_w):
    src = edge_index[0]
    dst = edge_index[1]
    pad_id = jnp.int32(NPAD - 1)
    src_r = jnp.concatenate(
        [src, jnp.full((EPAD - E,), pad_id, jnp.int32)]).reshape(NW, NCHUNK, CHUNK)
    dst_r = jnp.concatenate(
        [dst, jnp.full((EPAD - E,), pad_id, jnp.int32)]).reshape(NW, NCHUNK, CHUNK)

    h = jnp.zeros((NPAD, H), _f32).at[:N].set(x)
    mask = (jnp.arange(NPAD, dtype=jnp.int32) < N).astype(_f32)[:, None]

    ks = [5000, 2500, 1250]
    gs_list = []
    proj_list = []
    zero_gs = jnp.zeros((1, 2 * H), _f32)
    counts = jnp.full((NW, 16), NCHUNK, jnp.int32)
    out = None
    for i in range(3):
        for l in range(2):
            agg = _sc_agg(h, src_r, dst_r, counts)
            h = _tc_mlp(h, agg, mask, gin_W[i, l, 0], gin_b[i, l, 0],
                        gin_W[i, l, 1], gin_b[i, l, 1])
        g0 = gs_list[0] if i == 2 else zero_gs
        g1 = gs_list[1] if i == 2 else zero_gs
        h, mask, gs, pj, out = _tc_epilogue(
            ks[i], h, mask, pool_w[i], proj_W1[i], proj_b1[i], proj_W2[i],
            proj_b2[i], g0, g1)
        gs_list.append(gs)
        proj_list.append(pj)
        if i < 2:
            src_r, dst_r, counts = _sc_filter(src_r, dst_r, mask[:, 0], counts)

    return (out, gs_list[0], gs_list[1], gs_list[2],
            proj_list[0], proj_list[1], proj_list[2])


# speedup vs baseline: 6.9921x; 1.0058x over previous
import functools

import jax
import jax.numpy as jnp
from jax import lax
from jax.experimental import pallas as pl
from jax.experimental.pallas import tpu as pltpu
from jax.experimental.pallas import tpu_sc as plsc

N = 10000
E = 320000
H = 128
NPAD = 10240
NC, NS = 2, 16
NW = NC * NS
CHUNK = 128
NCHUNK = 80
EPAD = NW * NCHUNK * CHUNK
ROWS_PER_TILE = NPAD // NS
OUT_CHUNK = 64
NBUF = 2

_f32 = jnp.float32


def _agg_body(x_hbm, src_hbm, dst_hbm, cnt_hbm, out_hbm,
              sidx_v, didx_v, rows_v, zero_v, copy_v, cnt_v, agg_sh,
              isem, gsem):
    c = lax.axis_index("c")
    s = lax.axis_index("s")
    wid = s * NC + c

    pltpu.sync_copy(cnt_hbm.at[wid], cnt_v)
    nch = lax.reduce_max(cnt_v[...], axes=(0,))

    def idx_start(j, b):
        pltpu.async_copy(src_hbm.at[wid, j], sidx_v.at[b], isem.at[b])
        pltpu.async_copy(dst_hbm.at[wid, j], didx_v.at[b], isem.at[b])

    def idx_wait(j, b):
        pltpu.make_async_copy(src_hbm.at[wid, j], sidx_v.at[b], isem.at[b]).wait()
        pltpu.make_async_copy(dst_hbm.at[wid, j], didx_v.at[b], isem.at[b]).wait()

    def gather_start(b):
        pltpu.async_copy(x_hbm.at[sidx_v.at[b]], rows_v.at[b], gsem.at[b])

    def gather_wait(b):
        pltpu.make_async_copy(x_hbm.at[sidx_v.at[b]], rows_v.at[b],
                              gsem.at[b]).wait()

    idx_start(0, 0)
    idx_start(1, 1)
    idx_wait(0, 0)
    gather_start(0)

    for r in range(16):
        for j in range(H // 16):
            zero_v[r, pl.ds(j * 16, 16)] = jnp.zeros((16,), _f32)
    for i in range(ROWS_PER_TILE // 16):
        pltpu.sync_copy(zero_v, agg_sh.at[pl.ds(s * ROWS_PER_TILE + i * 16, 16)])
    plsc.subcore_barrier()

    def pair_step(j0, carry):
        j = j0 * 2
        gather_wait(0)
        pltpu.sync_copy(rows_v.at[0], agg_sh.at[didx_v.at[0]], add=True)

        @pl.when(j + 2 < nch)
        def _():
            idx_start(j + 2, 0)

        idx_wait(j + 1, 1)
        gather_start(1)

        gather_wait(1)
        pltpu.sync_copy(rows_v.at[1], agg_sh.at[didx_v.at[1]], add=True)

        @pl.when(j + 3 < nch)
        def _():
            idx_start(j + 3, 1)

        @pl.when(j + 2 < nch)
        def _():
            idx_wait(j + 2, 0)
            gather_start(0)

        return carry

    lax.fori_loop(0, nch // 2, pair_step, 0, unroll=False)
    plsc.subcore_barrier()

    base = s * ROWS_PER_TILE
    pltpu.sync_copy(agg_sh.at[pl.ds(base, ROWS_PER_TILE)],
                    out_hbm.at[c, pl.ds(base, ROWS_PER_TILE)])


@jax.jit
def _sc_agg(x_pad, src_r, dst_r, counts):
    mesh = plsc.VectorSubcoreMesh(core_axis_name="c", subcore_axis_name="s")
    return pl.kernel(
        _agg_body,
        compiler_params=pltpu.CompilerParams(needs_layout_passes=False),
        out_type=jax.ShapeDtypeStruct((NC, NPAD, H), _f32),
        mesh=mesh,
        scratch_types=[
            pltpu.VMEM((NBUF, CHUNK), jnp.int32),
            pltpu.VMEM((NBUF, CHUNK), jnp.int32),
            pltpu.VMEM((NBUF, CHUNK, H), _f32),
            pltpu.VMEM((16, H), _f32),
            pltpu.VMEM((OUT_CHUNK, H), _f32),
            pltpu.VMEM((16,), jnp.int32),
            pltpu.VMEM_SHARED((NPAD, H), _f32),
            pltpu.SemaphoreType.DMA((NBUF,)),
            pltpu.SemaphoreType.DMA((NBUF,)),
        ],
    )(x_pad, src_r, dst_r, counts)


_PADV = NPAD - 1
_VPC = CHUNK // 16


def _filter_body(src_hbm, dst_hbm, keep_hbm, cin_hbm, osrc_hbm, odst_hbm,
                 cnt_hbm, keep_v, sidx_v, didx_v, osrc_v, odst_v, cnt_v,
                 cin_v, pad_v):
    c = lax.axis_index("c")
    s = lax.axis_index("s")
    wid = s * NC + c

    pltpu.sync_copy(cin_hbm.at[wid], cin_v)
    nch_in = lax.reduce_max(cin_v[...], axes=(0,))
    pltpu.sync_copy(keep_hbm, keep_v)
    pltpu.sync_copy(src_hbm.at[wid], sidx_v)
    pltpu.sync_copy(dst_hbm.at[wid], didx_v)

    for i in range(16):
        pad_v[pl.ds(i * 16, 16)] = jnp.full((16,), _PADV, jnp.int32)

    def step(i, carry):
        cnt, cnt_vec = carry
        sv = sidx_v[pl.ds(i * 16, 16)]
        dv = didx_v[pl.ds(i * 16, 16)]
        ks = plsc.load_gather(keep_v, [sv >> 7, sv & 127])
        kd = plsc.load_gather(keep_v, [dv >> 7, dv & 127])
        m = (ks > 0.5) & (kd > 0.5)
        plsc.store_compressed(osrc_v.at[pl.ds(cnt, 16)], sv, mask=m)
        plsc.store_compressed(odst_v.at[pl.ds(cnt, 16)], dv, mask=m)
        npop = plsc.all_reduce_population_count(m)
        return cnt + lax.reduce_max(npop, axes=(0,)), cnt_vec + npop

    cnt, cnt_vec = lax.fori_loop(
        0, nch_in * _VPC, step,
        (jnp.int32(0), jnp.zeros((16,), jnp.int32)), unroll=False)

    def pad_step(i, carry):
        base = cnt + i * 16
        osrc_v[pl.ds(base, 16)] = pad_v[pl.ds(0, 16)]
        odst_v[pl.ds(base, 16)] = pad_v[pl.ds(0, 16)]
        return carry

    lax.fori_loop(0, 17, pad_step, 0, unroll=False)

    cnt_v[...] = jnp.maximum((cnt_vec + 255) // 256 * 2, 2)
    pltpu.sync_copy(cnt_v, cnt_hbm.at[wid])
    pltpu.sync_copy(osrc_v.at[pl.ds(0, NCHUNK * CHUNK)], osrc_hbm.at[wid])
    pltpu.sync_copy(odst_v.at[pl.ds(0, NCHUNK * CHUNK)], odst_hbm.at[wid])


@jax.jit
def _sc_filter(src_r, dst_r, keep, counts_in):
    mesh = plsc.VectorSubcoreMesh(core_axis_name="c", subcore_axis_name="s")
    src_f = src_r.reshape(NW, NCHUNK * CHUNK)
    dst_f = dst_r.reshape(NW, NCHUNK * CHUNK)
    osrc, odst, counts = pl.kernel(
        _filter_body,
        compiler_params=pltpu.CompilerParams(use_tc_tiling_on_sc=False,
                                             needs_layout_passes=False),
        out_type=(
            jax.ShapeDtypeStruct((NW, NCHUNK * CHUNK), jnp.int32),
            jax.ShapeDtypeStruct((NW, NCHUNK * CHUNK), jnp.int32),
            jax.ShapeDtypeStruct((NW, 16), jnp.int32),
        ),
        mesh=mesh,
        scratch_types=[
            pltpu.VMEM((NPAD // 128, 128), _f32),
            pltpu.VMEM((NCHUNK * CHUNK,), jnp.int32),
            pltpu.VMEM((NCHUNK * CHUNK,), jnp.int32),
            pltpu.VMEM((NCHUNK * CHUNK + 512,), jnp.int32),
            pltpu.VMEM((NCHUNK * CHUNK + 512,), jnp.int32),
            pltpu.VMEM((16,), jnp.int32),
            pltpu.VMEM((16,), jnp.int32),
            pltpu.VMEM((256,), jnp.int32),
        ],
    )(src_f, dst_f, keep.reshape(NPAD // 128, 128), counts_in)
    return (osrc.reshape(NW, NCHUNK, CHUNK), odst.reshape(NW, NCHUNK, CHUNK),
            counts)


_BLK = 512


def _mlp_body(x_ref, a_ref, m_ref, w0_ref, b0_ref, w1_ref, b1_ref, o_ref):
    h = x_ref[...] + a_ref[0] + a_ref[1]
    h = jnp.maximum(
        jax.lax.dot_general(h, w0_ref[...], (((1,), (0,)), ((), ())),
                            precision=jax.lax.Precision.DEFAULT) + b0_ref[...], 0.0)
    h = jax.lax.dot_general(h, w1_ref[...], (((1,), (0,)), ((), ())),
                            precision=jax.lax.Precision.DEFAULT) + b1_ref[...]
    o_ref[...] = jnp.maximum(h, 0.0) * m_ref[...]


@jax.jit
def _tc_mlp(x_pad, agg, mask_col, W0, b0, W1, b1):
    grid = NPAD // _BLK
    return pl.pallas_call(
        _mlp_body,
        grid=(grid,),
        in_specs=[
            pl.BlockSpec((_BLK, H), lambda i: (i, 0)),
            pl.BlockSpec((NC, _BLK, H), lambda i: (0, i, 0)),
            pl.BlockSpec((_BLK, 1), lambda i: (i, 0)),
            pl.BlockSpec((H, H), lambda i: (0, 0)),
            pl.BlockSpec((1, H), lambda i: (0, 0)),
            pl.BlockSpec((H, H), lambda i: (0, 0)),
            pl.BlockSpec((1, H), lambda i: (0, 0)),
        ],
        out_specs=pl.BlockSpec((_BLK, H), lambda i: (i, 0)),
        out_shape=jax.ShapeDtypeStruct((NPAD, H), _f32),
    )(x_pad, agg, mask_col, W0, b0[None, :], W1, b1[None, :])


def _epilogue_body(k, x_ref, m_ref, w_ref, p1_ref, c1_ref, p2_ref, c2_ref,
                   g0_ref, g1_ref,
                   xo_ref, mo_ref, gs_ref, pj_ref, out_ref):
    x = x_ref[...]
    alive = m_ref[...] > 0.5
    kf = jnp.float32(k)

    p = jnp.sum(x, axis=0, keepdims=True)
    ph = jnp.maximum(
        jax.lax.dot_general(p, p1_ref[...], (((1,), (0,)), ((), ())),
                            precision=jax.lax.Precision.DEFAULT) + c1_ref[...], 0.0)
    pj_ref[...] = jax.lax.dot_general(ph, p2_ref[...], (((1,), (0,)), ((), ())),
                                      precision=jax.lax.Precision.DEFAULT) + c2_ref[...]

    w = w_ref[...]
    inv_norm = jax.lax.rsqrt(jnp.sum(w * w))
    score = jax.lax.dot_general(x, w, (((1,), (0,)), ((), ())),
                                precision=jax.lax.Precision.DEFAULT) * inv_norm

    s = jax.lax.bitcast_convert_type(score, jnp.int32)
    key = s ^ jnp.where(s < 0, jnp.int32(0x7FFFFFFF), jnp.int32(0))

    def cnt_ge(t):
        return jnp.sum(jnp.where(alive & (key >= t), 1.0, 0.0))

    T0 = jnp.where(cnt_ge(jnp.int32(0)) >= kf, jnp.int32(0), jnp.int32(-(2**31)))

    def t_step(i, t):
        cand = t | (jnp.int32(1) << (30 - i))
        return jnp.where(cnt_ge(cand) >= kf, cand, t)

    T = lax.fori_loop(0, 31, t_step, T0)

    gt = alive & (key > T)
    eq = alive & (key == T)
    n_gt = jnp.sum(jnp.where(gt, 1.0, 0.0))
    need_eq = kf - n_gt

    idx = jax.lax.broadcasted_iota(jnp.int32, (NPAD, 1), 0)

    def i_step(i, t):
        cand = t | (jnp.int32(1) << (13 - i))
        cnt = jnp.sum(jnp.where(eq & (idx < cand), 1.0, 0.0))
        return jnp.where(cnt < need_eq, cand, t)

    tmax = lax.fori_loop(0, 14, i_step, jnp.int32(0))
    keep = gt | (eq & (idx <= tmax) & (need_eq > 0.0))

    gate = jnp.tanh(score)
    xg = jnp.where(keep, x * gate, 0.0)
    gmax = jnp.max(jnp.where(keep, xg, -jnp.inf), axis=0, keepdims=True)
    gmean = jnp.sum(xg, axis=0, keepdims=True) / kf
    gs = jnp.concatenate([gmax, gmean], axis=1)

    xo_ref[...] = xg
    mo_ref[...] = jnp.where(keep, 1.0, 0.0)
    gs_ref[...] = gs
    out_ref[...] = (jnp.maximum(g0_ref[...], 0.0) + jnp.maximum(g1_ref[...], 0.0)
                    + jnp.maximum(gs, 0.0))


@functools.partial(jax.jit, static_argnums=(0,))
def _tc_epilogue(k, x_pad, mask_col, pool_w, P1, c1, P2, c2, gs0, gs1):
    return pl.pallas_call(
        functools.partial(_epilogue_body, k),
        out_shape=(
            jax.ShapeDtypeStruct((NPAD, H), _f32),
            jax.ShapeDtypeStruct((NPAD, 1), _f32),
            jax.ShapeDtypeStruct((1, 2 * H), _f32),
            jax.ShapeDtypeStruct((1, H), _f32),
            jax.ShapeDtypeStruct((1, 2 * H), _f32),
        ),
    )(x_pad, mask_col, pool_w[:, None], P1, c1[None, :], P2, c2[None, :], gs0, gs1)


def kernel(x, edge_index, batch, gin_W, gin_b, proj_W1, proj_b1, proj_W2,
           proj_b2, pool_w):
    src = edge_index[0]
    dst = edge_index[1]
    pad_id = jnp.int32(NPAD - 1)
    src_r = jnp.concatenate(
        [src, jnp.full((EPAD - E,), pad_id, jnp.int32)]).reshape(NW, NCHUNK, CHUNK)
    dst_r = jnp.concatenate(
        [dst, jnp.full((EPAD - E,), pad_id, jnp.int32)]).reshape(NW, NCHUNK, CHUNK)

    h = jnp.zeros((NPAD, H), _f32).at[:N].set(x)
    mask = (jnp.arange(NPAD, dtype=jnp.int32) < N).astype(_f32)[:, None]

    ks = [5000, 2500, 1250]
    gs_list = []
    proj_list = []
    zero_gs = jnp.zeros((1, 2 * H), _f32)
    counts = jnp.full((NW, 16), NCHUNK, jnp.int32)
    out = None
    for i in range(3):
        for l in range(2):
            agg = _sc_agg(h, src_r, dst_r, counts)
            h = _tc_mlp(h, agg, mask, gin_W[i, l, 0], gin_b[i, l, 0],
                        gin_W[i, l, 1], gin_b[i, l, 1])
        g0 = gs_list[0] if i == 2 else zero_gs
        g1 = gs_list[1] if i == 2 else zero_gs
        h, mask, gs, pj, out = _tc_epilogue(
            ks[i], h, mask, pool_w[i], proj_W1[i], proj_b1[i], proj_W2[i],
            proj_b2[i], g0, g1)
        gs_list.append(gs)
        proj_list.append(pj)
        if i < 2:
            src_r, dst_r, counts = _sc_filter(src_r, dst_r, mask[:, 0], counts)

    return (out, gs_list[0], gs_list[1], gs_list[2],
            proj_list[0], proj_list[1], proj_list[2])


# speedup vs baseline: 7.0190x; 1.0038x over previous
import functools

import jax
import jax.numpy as jnp
from jax import lax
from jax.experimental import pallas as pl
from jax.experimental.pallas import tpu as pltpu
from jax.experimental.pallas import tpu_sc as plsc

N = 10000
E = 320000
H = 128
NPAD = 10240
NC, NS = 2, 16
NW = NC * NS
CHUNK = 128
NCHUNK = 80
EPAD = NW * NCHUNK * CHUNK
ROWS_PER_TILE = NPAD // NS
OUT_CHUNK = 64
NBUF = 2

_f32 = jnp.float32


def _agg_body(x_hbm, src_hbm, dst_hbm, cnt_hbm, out_hbm,
              sidx_v, didx_v, rows_v, cnt_v, agg_sh, isem, gsem, zsem):
    c = lax.axis_index("c")
    s = lax.axis_index("s")
    wid = s * NC + c

    pltpu.sync_copy(cnt_hbm.at[wid], cnt_v)
    nch = lax.reduce_max(cnt_v[...], axes=(0,))

    def idx_start(j, b):
        pltpu.async_copy(src_hbm.at[wid, j], sidx_v.at[b], isem.at[b])
        pltpu.async_copy(dst_hbm.at[wid, j], didx_v.at[b], isem.at[b])

    def idx_wait(j, b):
        pltpu.make_async_copy(src_hbm.at[wid, j], sidx_v.at[b], isem.at[b]).wait()
        pltpu.make_async_copy(dst_hbm.at[wid, j], didx_v.at[b], isem.at[b]).wait()

    def gather_start(b):
        pltpu.async_copy(x_hbm.at[sidx_v.at[b]], rows_v.at[b], gsem.at[b])

    def gather_wait(b):
        pltpu.make_async_copy(x_hbm.at[sidx_v.at[b]], rows_v.at[b],
                              gsem.at[b]).wait()

    def zstep(r, carry):
        for j in range(H // 16):
            rows_v[0, r, pl.ds(j * 16, 16)] = jnp.zeros((16,), _f32)
        return carry

    lax.fori_loop(0, CHUNK, zstep, 0, unroll=False)
    nz = ROWS_PER_TILE // CHUNK
    for i in range(nz):
        pltpu.async_copy(
            rows_v.at[0],
            agg_sh.at[pl.ds(s * ROWS_PER_TILE + i * CHUNK, CHUNK)], zsem)
    idx_start(0, 0)
    idx_start(1, 1)
    for i in range(nz):
        pltpu.make_async_copy(
            rows_v.at[0],
            agg_sh.at[pl.ds(s * ROWS_PER_TILE + i * CHUNK, CHUNK)], zsem).wait()
    idx_wait(0, 0)
    gather_start(0)
    plsc.subcore_barrier()

    def pair_step(j0, carry):
        j = j0 * 2
        gather_wait(0)
        pltpu.sync_copy(rows_v.at[0], agg_sh.at[didx_v.at[0]], add=True)

        @pl.when(j + 2 < nch)
        def _():
            idx_start(j + 2, 0)

        idx_wait(j + 1, 1)
        gather_start(1)

        gather_wait(1)
        pltpu.sync_copy(rows_v.at[1], agg_sh.at[didx_v.at[1]], add=True)

        @pl.when(j + 3 < nch)
        def _():
            idx_start(j + 3, 1)

        @pl.when(j + 2 < nch)
        def _():
            idx_wait(j + 2, 0)
            gather_start(0)

        return carry

    lax.fori_loop(0, nch // 2, pair_step, 0, unroll=False)
    plsc.subcore_barrier()

    base = s * ROWS_PER_TILE
    pltpu.sync_copy(agg_sh.at[pl.ds(base, ROWS_PER_TILE)],
                    out_hbm.at[c, pl.ds(base, ROWS_PER_TILE)])


@jax.jit
def _sc_agg(x_pad, src_r, dst_r, counts):
    mesh = plsc.VectorSubcoreMesh(core_axis_name="c", subcore_axis_name="s")
    return pl.kernel(
        _agg_body,
        compiler_params=pltpu.CompilerParams(needs_layout_passes=False),
        out_type=jax.ShapeDtypeStruct((NC, NPAD, H), _f32),
        mesh=mesh,
        scratch_types=[
            pltpu.VMEM((NBUF, CHUNK), jnp.int32),
            pltpu.VMEM((NBUF, CHUNK), jnp.int32),
            pltpu.VMEM((NBUF, CHUNK, H), _f32),
            pltpu.VMEM((16,), jnp.int32),
            pltpu.VMEM_SHARED((NPAD, H), _f32),
            pltpu.SemaphoreType.DMA((NBUF,)),
            pltpu.SemaphoreType.DMA((NBUF,)),
            pltpu.SemaphoreType.DMA,
        ],
    )(x_pad, src_r, dst_r, counts)


_PADV = NPAD - 1
_VPC = CHUNK // 16


def _filter_body(src_hbm, dst_hbm, keep_hbm, cin_hbm, osrc_hbm, odst_hbm,
                 cnt_hbm, keep_v, sidx_v, didx_v, osrc_v, odst_v, cnt_v,
                 cin_v, pad_v):
    c = lax.axis_index("c")
    s = lax.axis_index("s")
    wid = s * NC + c

    pltpu.sync_copy(cin_hbm.at[wid], cin_v)
    nch_in = lax.reduce_max(cin_v[...], axes=(0,))
    pltpu.sync_copy(keep_hbm, keep_v)
    pltpu.sync_copy(src_hbm.at[wid], sidx_v)
    pltpu.sync_copy(dst_hbm.at[wid], didx_v)

    for i in range(16):
        pad_v[pl.ds(i * 16, 16)] = jnp.full((16,), _PADV, jnp.int32)

    def step(i, carry):
        cnt, cnt_vec = carry
        sv = sidx_v[pl.ds(i * 16, 16)]
        dv = didx_v[pl.ds(i * 16, 16)]
        ks = plsc.load_gather(keep_v, [sv >> 7, sv & 127])
        kd = plsc.load_gather(keep_v, [dv >> 7, dv & 127])
        m = (ks > 0.5) & (kd > 0.5)
        plsc.store_compressed(osrc_v.at[pl.ds(cnt, 16)], sv, mask=m)
        plsc.store_compressed(odst_v.at[pl.ds(cnt, 16)], dv, mask=m)
        npop = plsc.all_reduce_population_count(m)
        return cnt + lax.reduce_max(npop, axes=(0,)), cnt_vec + npop

    cnt, cnt_vec = lax.fori_loop(
        0, nch_in * _VPC, step,
        (jnp.int32(0), jnp.zeros((16,), jnp.int32)), unroll=False)

    def pad_step(i, carry):
        base = cnt + i * 16
        osrc_v[pl.ds(base, 16)] = pad_v[pl.ds(0, 16)]
        odst_v[pl.ds(base, 16)] = pad_v[pl.ds(0, 16)]
        return carry

    lax.fori_loop(0, 17, pad_step, 0, unroll=False)

    cnt_v[...] = jnp.maximum((cnt_vec + 255) // 256 * 2, 2)
    pltpu.sync_copy(cnt_v, cnt_hbm.at[wid])
    pltpu.sync_copy(osrc_v.at[pl.ds(0, NCHUNK * CHUNK)], osrc_hbm.at[wid])
    pltpu.sync_copy(odst_v.at[pl.ds(0, NCHUNK * CHUNK)], odst_hbm.at[wid])


@jax.jit
def _sc_filter(src_r, dst_r, keep, counts_in):
    mesh = plsc.VectorSubcoreMesh(core_axis_name="c", subcore_axis_name="s")
    src_f = src_r.reshape(NW, NCHUNK * CHUNK)
    dst_f = dst_r.reshape(NW, NCHUNK * CHUNK)
    osrc, odst, counts = pl.kernel(
        _filter_body,
        compiler_params=pltpu.CompilerParams(use_tc_tiling_on_sc=False,
                                             needs_layout_passes=False),
        out_type=(
            jax.ShapeDtypeStruct((NW, NCHUNK * CHUNK), jnp.int32),
            jax.ShapeDtypeStruct((NW, NCHUNK * CHUNK), jnp.int32),
            jax.ShapeDtypeStruct((NW, 16), jnp.int32),
        ),
        mesh=mesh,
        scratch_types=[
            pltpu.VMEM((NPAD // 128, 128), _f32),
            pltpu.VMEM((NCHUNK * CHUNK,), jnp.int32),
            pltpu.VMEM((NCHUNK * CHUNK,), jnp.int32),
            pltpu.VMEM((NCHUNK * CHUNK + 512,), jnp.int32),
            pltpu.VMEM((NCHUNK * CHUNK + 512,), jnp.int32),
            pltpu.VMEM((16,), jnp.int32),
            pltpu.VMEM((16,), jnp.int32),
            pltpu.VMEM((256,), jnp.int32),
        ],
    )(src_f, dst_f, keep.reshape(NPAD // 128, 128), counts_in)
    return (osrc.reshape(NW, NCHUNK, CHUNK), odst.reshape(NW, NCHUNK, CHUNK),
            counts)


_BLK = 512


def _mlp_body(x_ref, a_ref, m_ref, w0_ref, b0_ref, w1_ref, b1_ref, o_ref):
    h = x_ref[...] + a_ref[0] + a_ref[1]
    h = jnp.maximum(
        jax.lax.dot_general(h, w0_ref[...], (((1,), (0,)), ((), ())),
                            precision=jax.lax.Precision.DEFAULT) + b0_ref[...], 0.0)
    h = jax.lax.dot_general(h, w1_ref[...], (((1,), (0,)), ((), ())),
                            precision=jax.lax.Precision.DEFAULT) + b1_ref[...]
    o_ref[...] = jnp.maximum(h, 0.0) * m_ref[...]


@jax.jit
def _tc_mlp(x_pad, agg, mask_col, W0, b0, W1, b1):
    grid = NPAD // _BLK
    return pl.pallas_call(
        _mlp_body,
        grid=(grid,),
        in_specs=[
            pl.BlockSpec((_BLK, H), lambda i: (i, 0)),
            pl.BlockSpec((NC, _BLK, H), lambda i: (0, i, 0)),
            pl.BlockSpec((_BLK, 1), lambda i: (i, 0)),
            pl.BlockSpec((H, H), lambda i: (0, 0)),
            pl.BlockSpec((1, H), lambda i: (0, 0)),
            pl.BlockSpec((H, H), lambda i: (0, 0)),
            pl.BlockSpec((1, H), lambda i: (0, 0)),
        ],
        out_specs=pl.BlockSpec((_BLK, H), lambda i: (i, 0)),
        out_shape=jax.ShapeDtypeStruct((NPAD, H), _f32),
    )(x_pad, agg, mask_col, W0, b0[None, :], W1, b1[None, :])


def _epilogue_body(k, x_ref, m_ref, w_ref, p1_ref, c1_ref, p2_ref, c2_ref,
                   g0_ref, g1_ref,
                   xo_ref, mo_ref, gs_ref, pj_ref, out_ref):
    x = x_ref[...]
    alive = m_ref[...] > 0.5
    kf = jnp.float32(k)

    p = jnp.sum(x, axis=0, keepdims=True)
    ph = jnp.maximum(
        jax.lax.dot_general(p, p1_ref[...], (((1,), (0,)), ((), ())),
                            precision=jax.lax.Precision.DEFAULT) + c1_ref[...], 0.0)
    pj_ref[...] = jax.lax.dot_general(ph, p2_ref[...], (((1,), (0,)), ((), ())),
                                      precision=jax.lax.Precision.DEFAULT) + c2_ref[...]

    w = w_ref[...]
    inv_norm = jax.lax.rsqrt(jnp.sum(w * w))
    score = jax.lax.dot_general(x, w, (((1,), (0,)), ((), ())),
                                precision=jax.lax.Precision.DEFAULT) * inv_norm

    s = jax.lax.bitcast_convert_type(score, jnp.int32)
    key = s ^ jnp.where(s < 0, jnp.int32(0x7FFFFFFF), jnp.int32(0))

    def cnt_ge(t):
        return jnp.sum(jnp.where(alive & (key >= t), 1.0, 0.0))

    T0 = jnp.where(cnt_ge(jnp.int32(0)) >= kf, jnp.int32(0), jnp.int32(-(2**31)))

    def t_step(i, t):
        cand = t | (jnp.int32(1) << (30 - i))
        return jnp.where(cnt_ge(cand) >= kf, cand, t)

    T = lax.fori_loop(0, 31, t_step, T0)

    gt = alive & (key > T)
    eq = alive & (key == T)
    n_gt = jnp.sum(jnp.where(gt, 1.0, 0.0))
    need_eq = kf - n_gt

    idx = jax.lax.broadcasted_iota(jnp.int32, (NPAD, 1), 0)

    def i_step(i, t):
        cand = t | (jnp.int32(1) << (13 - i))
        cnt = jnp.sum(jnp.where(eq & (idx < cand), 1.0, 0.0))
        return jnp.where(cnt < need_eq, cand, t)

    tmax = lax.fori_loop(0, 14, i_step, jnp.int32(0))
    keep = gt | (eq & (idx <= tmax) & (need_eq > 0.0))

    gate = jnp.tanh(score)
    xg = jnp.where(keep, x * gate, 0.0)
    gmax = jnp.max(jnp.where(keep, xg, -jnp.inf), axis=0, keepdims=True)
    gmean = jnp.sum(xg, axis=0, keepdims=True) / kf
    gs = jnp.concatenate([gmax, gmean], axis=1)

    xo_ref[...] = xg
    mo_ref[...] = jnp.where(keep, 1.0, 0.0)
    gs_ref[...] = gs
    out_ref[...] = (jnp.maximum(g0_ref[...], 0.0) + jnp.maximum(g1_ref[...], 0.0)
                    + jnp.maximum(gs, 0.0))


@functools.partial(jax.jit, static_argnums=(0,))
def _tc_epilogue(k, x_pad, mask_col, pool_w, P1, c1, P2, c2, gs0, gs1):
    return pl.pallas_call(
        functools.partial(_epilogue_body, k),
        out_shape=(
            jax.ShapeDtypeStruct((NPAD, H), _f32),
            jax.ShapeDtypeStruct((NPAD, 1), _f32),
            jax.ShapeDtypeStruct((1, 2 * H), _f32),
            jax.ShapeDtypeStruct((1, H), _f32),
            jax.ShapeDtypeStruct((1, 2 * H), _f32),
        ),
    )(x_pad, mask_col, pool_w[:, None], P1, c1[None, :], P2, c2[None, :], gs0, gs1)


def kernel(x, edge_index, batch, gin_W, gin_b, proj_W1, proj_b1, proj_W2,
           proj_b2, pool_w):
    src = edge_index[0]
    dst = edge_index[1]
    pad_id = jnp.int32(NPAD - 1)
    src_r = jnp.concatenate(
        [src, jnp.full((EPAD - E,), pad_id, jnp.int32)]).reshape(NW, NCHUNK, CHUNK)
    dst_r = jnp.concatenate(
        [dst, jnp.full((EPAD - E,), pad_id, jnp.int32)]).reshape(NW, NCHUNK, CHUNK)

    h = jnp.zeros((NPAD, H), _f32).at[:N].set(x)
    mask = (jnp.arange(NPAD, dtype=jnp.int32) < N).astype(_f32)[:, None]

    ks = [5000, 2500, 1250]
    gs_list = []
    proj_list = []
    zero_gs = jnp.zeros((1, 2 * H), _f32)
    counts = jnp.full((NW, 16), NCHUNK, jnp.int32)
    out = None
    for i in range(3):
        for l in range(2):
            agg = _sc_agg(h, src_r, dst_r, counts)
            h = _tc_mlp(h, agg, mask, gin_W[i, l, 0], gin_b[i, l, 0],
                        gin_W[i, l, 1], gin_b[i, l, 1])
        g0 = gs_list[0] if i == 2 else zero_gs
        g1 = gs_list[1] if i == 2 else zero_gs
        h, mask, gs, pj, out = _tc_epilogue(
            ks[i], h, mask, pool_w[i], proj_W1[i], proj_b1[i], proj_W2[i],
            proj_b2[i], g0, g1)
        gs_list.append(gs)
        proj_list.append(pj)
        if i < 2:
            src_r, dst_r, counts = _sc_filter(src_r, dst_r, mask[:, 0], counts)

    return (out, gs_list[0], gs_list[1], gs_list[2],
            proj_list[0], proj_list[1], proj_list[2])


# speedup vs baseline: 7.3874x; 1.0525x over previous
import functools

import jax
import jax.numpy as jnp
from jax import lax
from jax.experimental import pallas as pl
from jax.experimental.pallas import tpu as pltpu
from jax.experimental.pallas import tpu_sc as plsc

N = 10000
E = 320000
H = 128
NPAD = 10240
NC, NS = 2, 16
NW = NC * NS
CHUNK = 128
NCHUNK = 80
EPAD = NW * NCHUNK * CHUNK
ROWS_PER_TILE = NPAD // NS
OUT_CHUNK = 64
NBUF = 2

_f32 = jnp.float32


def _agg_body(x_hbm, src_hbm, dst_hbm, cnt_hbm, out_hbm,
              sidx_v, didx_v, rows_v, cnt_v, agg_sh, isem, gsem, zsem):
    c = lax.axis_index("c")
    s = lax.axis_index("s")
    wid = s * NC + c

    pltpu.sync_copy(cnt_hbm.at[wid], cnt_v)
    nch = lax.reduce_max(cnt_v[...], axes=(0,))

    def idx_start(j, b):
        pltpu.async_copy(src_hbm.at[wid, j], sidx_v.at[b], isem.at[b])
        pltpu.async_copy(dst_hbm.at[wid, j], didx_v.at[b], isem.at[b])

    def idx_wait(j, b):
        pltpu.make_async_copy(src_hbm.at[wid, j], sidx_v.at[b], isem.at[b]).wait()
        pltpu.make_async_copy(dst_hbm.at[wid, j], didx_v.at[b], isem.at[b]).wait()

    def gather_start(b):
        pltpu.async_copy(x_hbm.at[sidx_v.at[b]], rows_v.at[b], gsem.at[b])

    def gather_wait(b):
        pltpu.make_async_copy(x_hbm.at[sidx_v.at[b]], rows_v.at[b],
                              gsem.at[b]).wait()

    def zstep(r, carry):
        for j in range(H // 16):
            rows_v[0, r, pl.ds(j * 16, 16)] = jnp.zeros((16,), _f32)
        return carry

    lax.fori_loop(0, CHUNK, zstep, 0, unroll=False)
    nz = ROWS_PER_TILE // CHUNK
    for i in range(nz):
        pltpu.async_copy(
            rows_v.at[0],
            agg_sh.at[pl.ds(s * ROWS_PER_TILE + i * CHUNK, CHUNK)], zsem)
    idx_start(0, 0)
    idx_start(1, 1)
    for i in range(nz):
        pltpu.make_async_copy(
            rows_v.at[0],
            agg_sh.at[pl.ds(s * ROWS_PER_TILE + i * CHUNK, CHUNK)], zsem).wait()
    idx_wait(0, 0)
    gather_start(0)
    plsc.subcore_barrier()

    def pair_step(j0, carry):
        j = j0 * 2
        gather_wait(0)
        pltpu.sync_copy(rows_v.at[0], agg_sh.at[didx_v.at[0]], add=True)

        @pl.when(j + 2 < nch)
        def _():
            idx_start(j + 2, 0)

        idx_wait(j + 1, 1)
        gather_start(1)

        gather_wait(1)
        pltpu.sync_copy(rows_v.at[1], agg_sh.at[didx_v.at[1]], add=True)

        @pl.when(j + 3 < nch)
        def _():
            idx_start(j + 3, 1)

        @pl.when(j + 2 < nch)
        def _():
            idx_wait(j + 2, 0)
            gather_start(0)

        return carry

    lax.fori_loop(0, nch // 2, pair_step, 0, unroll=False)
    plsc.subcore_barrier()

    base = s * ROWS_PER_TILE
    pltpu.sync_copy(agg_sh.at[pl.ds(base, ROWS_PER_TILE)],
                    out_hbm.at[c, pl.ds(base, ROWS_PER_TILE)])


@jax.jit
def _sc_agg(x_pad, src_r, dst_r, counts):
    mesh = plsc.VectorSubcoreMesh(core_axis_name="c", subcore_axis_name="s")
    return pl.kernel(
        _agg_body,
        compiler_params=pltpu.CompilerParams(needs_layout_passes=False),
        out_type=jax.ShapeDtypeStruct((NC, NPAD, H), _f32),
        mesh=mesh,
        scratch_types=[
            pltpu.VMEM((NBUF, CHUNK), jnp.int32),
            pltpu.VMEM((NBUF, CHUNK), jnp.int32),
            pltpu.VMEM((NBUF, CHUNK, H), _f32),
            pltpu.VMEM((16,), jnp.int32),
            pltpu.VMEM_SHARED((NPAD, H), _f32),
            pltpu.SemaphoreType.DMA((NBUF,)),
            pltpu.SemaphoreType.DMA((NBUF,)),
            pltpu.SemaphoreType.DMA,
        ],
    )(x_pad, src_r, dst_r, counts)


_PADV = NPAD - 1
_VPC = CHUNK // 16


def _filter_body(src_hbm, dst_hbm, keep_hbm, cin_hbm, osrc_hbm, odst_hbm,
                 cnt_hbm, keep_v, sidx_v, didx_v, osrc_v, odst_v, cnt_v,
                 cin_v, pad_v):
    c = lax.axis_index("c")
    s = lax.axis_index("s")
    wid = s * NC + c

    pltpu.sync_copy(cin_hbm.at[wid], cin_v)
    nch_in = lax.reduce_max(cin_v[...], axes=(0,))
    pltpu.sync_copy(keep_hbm, keep_v)
    pltpu.sync_copy(src_hbm.at[wid], sidx_v)
    pltpu.sync_copy(dst_hbm.at[wid], didx_v)

    for i in range(16):
        pad_v[pl.ds(i * 16, 16)] = jnp.full((16,), _PADV, jnp.int32)

    def step(i, carry):
        cnt, cnt_vec = carry
        sv = sidx_v[pl.ds(i * 16, 16)]
        dv = didx_v[pl.ds(i * 16, 16)]
        ks = plsc.load_gather(keep_v, [sv >> 7, sv & 127])
        kd = plsc.load_gather(keep_v, [dv >> 7, dv & 127])
        m = (ks > 0.5) & (kd > 0.5)
        plsc.store_compressed(osrc_v.at[pl.ds(cnt, 16)], sv, mask=m)
        plsc.store_compressed(odst_v.at[pl.ds(cnt, 16)], dv, mask=m)
        npop = plsc.all_reduce_population_count(m)
        return cnt + lax.reduce_max(npop, axes=(0,)), cnt_vec + npop

    cnt, cnt_vec = lax.fori_loop(
        0, nch_in * _VPC, step,
        (jnp.int32(0), jnp.zeros((16,), jnp.int32)), unroll=False)

    def pad_step(i, carry):
        base = cnt + i * 16
        osrc_v[pl.ds(base, 16)] = pad_v[pl.ds(0, 16)]
        odst_v[pl.ds(base, 16)] = pad_v[pl.ds(0, 16)]
        return carry

    lax.fori_loop(0, 17, pad_step, 0, unroll=False)

    cnt_v[...] = jnp.maximum((cnt_vec + 255) // 256 * 2, 2)
    pltpu.sync_copy(cnt_v, cnt_hbm.at[wid])
    pltpu.sync_copy(osrc_v.at[pl.ds(0, NCHUNK * CHUNK)], osrc_hbm.at[wid])
    pltpu.sync_copy(odst_v.at[pl.ds(0, NCHUNK * CHUNK)], odst_hbm.at[wid])


@jax.jit
def _sc_filter(src_r, dst_r, keep, counts_in):
    mesh = plsc.VectorSubcoreMesh(core_axis_name="c", subcore_axis_name="s")
    src_f = src_r.reshape(NW, NCHUNK * CHUNK)
    dst_f = dst_r.reshape(NW, NCHUNK * CHUNK)
    osrc, odst, counts = pl.kernel(
        _filter_body,
        compiler_params=pltpu.CompilerParams(use_tc_tiling_on_sc=False,
                                             needs_layout_passes=False),
        out_type=(
            jax.ShapeDtypeStruct((NW, NCHUNK * CHUNK), jnp.int32),
            jax.ShapeDtypeStruct((NW, NCHUNK * CHUNK), jnp.int32),
            jax.ShapeDtypeStruct((NW, 16), jnp.int32),
        ),
        mesh=mesh,
        scratch_types=[
            pltpu.VMEM((NPAD // 128, 128), _f32),
            pltpu.VMEM((NCHUNK * CHUNK,), jnp.int32),
            pltpu.VMEM((NCHUNK * CHUNK,), jnp.int32),
            pltpu.VMEM((NCHUNK * CHUNK + 512,), jnp.int32),
            pltpu.VMEM((NCHUNK * CHUNK + 512,), jnp.int32),
            pltpu.VMEM((16,), jnp.int32),
            pltpu.VMEM((16,), jnp.int32),
            pltpu.VMEM((256,), jnp.int32),
        ],
    )(src_f, dst_f, keep.reshape(NPAD // 128, 128), counts_in)
    return (osrc.reshape(NW, NCHUNK, CHUNK), odst.reshape(NW, NCHUNK, CHUNK),
            counts)


CHUNK_A = 64


def _fagg_body(x_hbm, src_hbm, dst_hbm, keepb_hbm, cin_hbm,
               out_hbm, osrc_hbm, odst_hbm, cnt_hbm,
               keepb_v, fsrc_v, fdst_v, osrc_v, odst_v, didx2_v, rows_v,
               cnt_v, cin_v, agg_sh, fisem, gsem, zsem):
    c = lax.axis_index("c")
    s = lax.axis_index("s")
    wid = s * NC + c

    pltpu.sync_copy(cin_hbm.at[wid], cin_v)
    nch_in = lax.reduce_max(cin_v[...], axes=(0,))
    pltpu.sync_copy(keepb_hbm, keepb_v)

    def fidx_start(j, b):
        pltpu.async_copy(src_hbm.at[wid, j], fsrc_v.at[b], fisem.at[b])
        pltpu.async_copy(dst_hbm.at[wid, j], fdst_v.at[b], fisem.at[b])

    def fidx_wait(j, b):
        pltpu.make_async_copy(src_hbm.at[wid, j], fsrc_v.at[b],
                              fisem.at[b]).wait()
        pltpu.make_async_copy(dst_hbm.at[wid, j], fdst_v.at[b],
                              fisem.at[b]).wait()

    def keep_of(v):
        kw = plsc.load_gather(keepb_v, [v >> 9, (v >> 5) & 15])
        return jax.lax.shift_right_logical(kw, v & 31) & 1

    fidx_start(0, 0)
    fidx_start(1, 1)

    def fpair(j0, carry):
        cnt, cnt_vec = carry
        for b in range(2):
            j = j0 * 2 + b
            fidx_wait(j, b)
            for v in range(_VPC):
                sv = fsrc_v[b, pl.ds(v * 16, 16)]
                dv = fdst_v[b, pl.ds(v * 16, 16)]
                m = (keep_of(sv) & keep_of(dv)) > 0
                plsc.store_compressed(osrc_v.at[pl.ds(cnt, 16)], sv, mask=m)
                plsc.store_compressed(odst_v.at[pl.ds(cnt, 16)], dv, mask=m)
                npop = plsc.all_reduce_population_count(m)
                cnt = cnt + lax.reduce_max(npop, axes=(0,))
                cnt_vec = cnt_vec + npop

            @pl.when(j + 2 < nch_in)
            def _():
                fidx_start(j + 2, b)

        return cnt, cnt_vec

    cnt, cnt_vec = lax.fori_loop(
        0, nch_in // 2, fpair,
        (jnp.int32(0), jnp.zeros((16,), jnp.int32)), unroll=False)

    def pad_step(i, carry):
        base = cnt + i * 16
        osrc_v[pl.ds(base, 16)] = jnp.full((16,), _PADV, jnp.int32)
        odst_v[pl.ds(base, 16)] = jnp.full((16,), _PADV, jnp.int32)
        return carry

    lax.fori_loop(0, 17, pad_step, 0, unroll=False)

    cnt_v[...] = jnp.maximum((cnt_vec + 255) // 256 * 2, 2)
    nch_out = lax.reduce_max(cnt_v[...], axes=(0,))
    pltpu.sync_copy(cnt_v, cnt_hbm.at[wid])
    pltpu.sync_copy(osrc_v.at[pl.ds(0, NCHUNK * CHUNK)], osrc_hbm.at[wid])
    pltpu.sync_copy(odst_v.at[pl.ds(0, NCHUNK * CHUNK)], odst_hbm.at[wid])

    def zstep(r, carry):
        for j in range(H // 16):
            rows_v[0, r, pl.ds(j * 16, 16)] = jnp.zeros((16,), _f32)
        return carry

    lax.fori_loop(0, CHUNK_A, zstep, 0, unroll=False)
    nz = ROWS_PER_TILE // CHUNK_A
    for i in range(nz):
        pltpu.async_copy(
            rows_v.at[0],
            agg_sh.at[pl.ds(s * ROWS_PER_TILE + i * CHUNK_A, CHUNK_A)], zsem)
    for i in range(nz):
        pltpu.make_async_copy(
            rows_v.at[0],
            agg_sh.at[pl.ds(s * ROWS_PER_TILE + i * CHUNK_A, CHUNK_A)],
            zsem).wait()
    plsc.subcore_barrier()

    def gstart(j, b):
        for v in range(CHUNK_A // 16):
            didx2_v[b, pl.ds(v * 16, 16)] = odst_v[pl.ds(j * CHUNK_A + v * 16,
                                                         16)]
        pltpu.async_copy(x_hbm.at[osrc_v.at[pl.ds(j * CHUNK_A, CHUNK_A)]],
                         rows_v.at[b], gsem.at[b])

    def gwait(j, b):
        pltpu.make_async_copy(
            x_hbm.at[osrc_v.at[pl.ds(j * CHUNK_A, CHUNK_A)]],
            rows_v.at[b], gsem.at[b]).wait()

    nlim = nch_out * (CHUNK // CHUNK_A)
    gstart(0, 0)
    gstart(1, 1)

    def apair(j0, carry):
        for b in range(2):
            j = j0 * 2 + b
            gwait(j, b)
            pltpu.sync_copy(rows_v.at[b], agg_sh.at[didx2_v.at[b]], add=True)

            @pl.when(j + 2 < nlim)
            def _():
                gstart(j + 2, b)

        return carry

    lax.fori_loop(0, nlim // 2, apair, 0, unroll=False)
    plsc.subcore_barrier()

    base = s * ROWS_PER_TILE
    pltpu.sync_copy(agg_sh.at[pl.ds(base, ROWS_PER_TILE)],
                    out_hbm.at[c, pl.ds(base, ROWS_PER_TILE)])


@jax.jit
def _sc_fagg(x_pad, src_r, dst_r, keepb, counts_in):
    mesh = plsc.VectorSubcoreMesh(core_axis_name="c", subcore_axis_name="s")
    return pl.kernel(
        _fagg_body,
        compiler_params=pltpu.CompilerParams(use_tc_tiling_on_sc=False,
                                             needs_layout_passes=False),
        out_type=(
            jax.ShapeDtypeStruct((NC, NPAD, H), _f32),
            jax.ShapeDtypeStruct((NW, NCHUNK * CHUNK), jnp.int32),
            jax.ShapeDtypeStruct((NW, NCHUNK * CHUNK), jnp.int32),
            jax.ShapeDtypeStruct((NW, 16), jnp.int32),
        ),
        mesh=mesh,
        scratch_types=[
            pltpu.VMEM((NPAD // 512, 16), jnp.int32),
            pltpu.VMEM((2, CHUNK), jnp.int32),
            pltpu.VMEM((2, CHUNK), jnp.int32),
            pltpu.VMEM((NCHUNK * CHUNK + 512,), jnp.int32),
            pltpu.VMEM((NCHUNK * CHUNK + 512,), jnp.int32),
            pltpu.VMEM((2, CHUNK_A), jnp.int32),
            pltpu.VMEM((2, CHUNK_A, H), _f32),
            pltpu.VMEM((16,), jnp.int32),
            pltpu.VMEM((16,), jnp.int32),
            pltpu.VMEM_SHARED((NPAD, H), _f32),
            pltpu.SemaphoreType.DMA((2,)),
            pltpu.SemaphoreType.DMA((2,)),
            pltpu.SemaphoreType.DMA,
        ],
    )(x_pad, src_r, dst_r, keepb, counts_in)


_BLK = 512


def _mlp_body(x_ref, a_ref, m_ref, w0_ref, b0_ref, w1_ref, b1_ref, o_ref):
    h = x_ref[...] + a_ref[0] + a_ref[1]
    h = jnp.maximum(
        jax.lax.dot_general(h, w0_ref[...], (((1,), (0,)), ((), ())),
                            precision=jax.lax.Precision.DEFAULT) + b0_ref[...], 0.0)
    h = jax.lax.dot_general(h, w1_ref[...], (((1,), (0,)), ((), ())),
                            precision=jax.lax.Precision.DEFAULT) + b1_ref[...]
    o_ref[...] = jnp.maximum(h, 0.0) * m_ref[...]


@jax.jit
def _tc_mlp(x_pad, agg, mask_col, W0, b0, W1, b1):
    grid = NPAD // _BLK
    return pl.pallas_call(
        _mlp_body,
        grid=(grid,),
        in_specs=[
            pl.BlockSpec((_BLK, H), lambda i: (i, 0)),
            pl.BlockSpec((NC, _BLK, H), lambda i: (0, i, 0)),
            pl.BlockSpec((_BLK, 1), lambda i: (i, 0)),
            pl.BlockSpec((H, H), lambda i: (0, 0)),
            pl.BlockSpec((1, H), lambda i: (0, 0)),
            pl.BlockSpec((H, H), lambda i: (0, 0)),
            pl.BlockSpec((1, H), lambda i: (0, 0)),
        ],
        out_specs=pl.BlockSpec((_BLK, H), lambda i: (i, 0)),
        out_shape=jax.ShapeDtypeStruct((NPAD, H), _f32),
    )(x_pad, agg, mask_col, W0, b0[None, :], W1, b1[None, :])


def _epilogue_body(k, x_ref, m_ref, w_ref, p1_ref, c1_ref, p2_ref, c2_ref,
                   g0_ref, g1_ref,
                   xo_ref, mo_ref, gs_ref, pj_ref, out_ref):
    x = x_ref[...]
    alive = m_ref[...] > 0.5
    kf = jnp.float32(k)

    p = jnp.sum(x, axis=0, keepdims=True)
    ph = jnp.maximum(
        jax.lax.dot_general(p, p1_ref[...], (((1,), (0,)), ((), ())),
                            precision=jax.lax.Precision.DEFAULT) + c1_ref[...], 0.0)
    pj_ref[...] = jax.lax.dot_general(ph, p2_ref[...], (((1,), (0,)), ((), ())),
                                      precision=jax.lax.Precision.DEFAULT) + c2_ref[...]

    w = w_ref[...]
    raw = jax.lax.dot_general(x, w, (((1,), (0,)), ((), ())),
                              precision=jax.lax.Precision.DEFAULT)
    score = raw / jnp.sqrt(jnp.sum(w * w))

    s = jax.lax.bitcast_convert_type(raw, jnp.int32)
    key = s ^ jnp.where(s < 0, jnp.int32(0x7FFFFFFF), jnp.int32(0))

    def cnt_ge(t):
        return jnp.sum(jnp.where(alive & (key >= t), 1.0, 0.0))

    T0 = jnp.where(cnt_ge(jnp.int32(0)) >= kf, jnp.int32(0), jnp.int32(-(2**31)))

    def t_step(i, t):
        cand = t | (jnp.int32(1) << (30 - i))
        return jnp.where(cnt_ge(cand) >= kf, cand, t)

    T = lax.fori_loop(0, 31, t_step, T0)

    gt = alive & (key > T)
    eq = alive & (key == T)
    n_gt = jnp.sum(jnp.where(gt, 1.0, 0.0))
    need_eq = kf - n_gt

    idx = jax.lax.broadcasted_iota(jnp.int32, (NPAD, 1), 0)

    def i_step(i, t):
        cand = t | (jnp.int32(1) << (13 - i))
        cnt = jnp.sum(jnp.where(eq & (idx < cand), 1.0, 0.0))
        return jnp.where(cnt < need_eq, cand, t)

    tmax = lax.fori_loop(0, 14, i_step, jnp.int32(0))
    keep = gt | (eq & (idx <= tmax) & (need_eq > 0.0))

    gate = jnp.tanh(score)
    xg = jnp.where(keep, x * gate, 0.0)
    gmax = jnp.max(jnp.where(keep, xg, -jnp.inf), axis=0, keepdims=True)
    gmean = jnp.sum(xg, axis=0, keepdims=True) / kf
    gs = jnp.concatenate([gmax, gmean], axis=1)

    xo_ref[...] = xg
    mo_ref[...] = jnp.where(keep, 1.0, 0.0)
    gs_ref[...] = gs
    out_ref[...] = (jnp.maximum(g0_ref[...], 0.0) + jnp.maximum(g1_ref[...], 0.0)
                    + jnp.maximum(gs, 0.0))


@functools.partial(jax.jit, static_argnums=(0,))
def _tc_epilogue(k, x_pad, mask_col, pool_w, P1, c1, P2, c2, gs0, gs1):
    return pl.pallas_call(
        functools.partial(_epilogue_body, k),
        out_shape=(
            jax.ShapeDtypeStruct((NPAD, H), _f32),
            jax.ShapeDtypeStruct((NPAD, 1), _f32),
            jax.ShapeDtypeStruct((1, 2 * H), _f32),
            jax.ShapeDtypeStruct((1, H), _f32),
            jax.ShapeDtypeStruct((1, 2 * H), _f32),
        ),
    )(x_pad, mask_col, pool_w[:, None], P1, c1[None, :], P2, c2[None, :], gs0, gs1)


def kernel(x, edge_index, batch, gin_W, gin_b, proj_W1, proj_b1, proj_W2,
           proj_b2, pool_w):
    src = edge_index[0]
    dst = edge_index[1]
    pad_id = jnp.int32(NPAD - 1)
    src_r = jnp.concatenate(
        [src, jnp.full((EPAD - E,), pad_id, jnp.int32)]).reshape(NW, NCHUNK, CHUNK)
    dst_r = jnp.concatenate(
        [dst, jnp.full((EPAD - E,), pad_id, jnp.int32)]).reshape(NW, NCHUNK, CHUNK)

    h = jnp.zeros((NPAD, H), _f32).at[:N].set(x)
    mask = (jnp.arange(NPAD, dtype=jnp.int32) < N).astype(_f32)[:, None]

    ks = [5000, 2500, 1250]
    gs_list = []
    proj_list = []
    zero_gs = jnp.zeros((1, 2 * H), _f32)
    counts = jnp.full((NW, 16), NCHUNK, jnp.int32)
    w16 = (2.0 ** jnp.arange(16)).astype(_f32)
    keepb = None
    out = None
    for i in range(3):
        for l in range(2):
            if l == 0 and i > 0:
                agg, osrc, odst, counts = _sc_fagg(h, src_r, dst_r, keepb,
                                                   counts)
                src_r = osrc.reshape(NW, NCHUNK, CHUNK)
                dst_r = odst.reshape(NW, NCHUNK, CHUNK)
            else:
                agg = _sc_agg(h, src_r, dst_r, counts)
            h = _tc_mlp(h, agg, mask, gin_W[i, l, 0], gin_b[i, l, 0],
                        gin_W[i, l, 1], gin_b[i, l, 1])
        g0 = gs_list[0] if i == 2 else zero_gs
        g1 = gs_list[1] if i == 2 else zero_gs
        h, mask, gs, pj, out = _tc_epilogue(
            ks[i], h, mask, pool_w[i], proj_W1[i], proj_b1[i], proj_W2[i],
            proj_b2[i], g0, g1)
        gs_list.append(gs)
        proj_list.append(pj)
        if i < 2:
            m2 = mask[:, 0].reshape(NPAD // 32, 32)
            lo = jnp.sum(m2[:, :16] * w16, axis=1).astype(jnp.int32)
            hi = jnp.sum(m2[:, 16:] * w16, axis=1).astype(jnp.int32)
            keepb = (lo | (hi << 16)).reshape(NPAD // 512, 16)

    return (out, gs_list[0], gs_list[1], gs_list[2],
            proj_list[0], proj_list[1], proj_list[2])
